# baseline (speedup 1.0000x reference)
import functools

import jax
import jax.numpy as jnp
import numpy as np
from jax import lax
from jax.experimental import pallas as pl
from jax.experimental.pallas import tpu as pltpu
from jax.experimental.pallas import tpu_sc as plsc


def _pos_encoding(seq_len, d_model):
    pos = np.arange(seq_len, dtype=np.float32)[:, None]
    two_i = np.arange(0, d_model, 2, dtype=np.float32)
    div = np.power(np.float32(10000.0), two_i / np.float32(d_model))
    enc = np.zeros((seq_len, d_model), dtype=np.float32)
    enc[:, 0::2] = np.sin(pos / div)
    enc[:, 1::2] = np.cos(pos / div)
    return jnp.asarray(enc)


@functools.lru_cache(maxsize=None)
def _make_sc_kernel(B, S, D):
    info = plsc.get_sparse_core_info()
    NC, NS, L = info.num_cores, info.num_subcores, info.num_lanes
    NW = NC * NS
    N = B * S
    assert N % NW == 0
    per_w = N // NW
    CHUNK = 40
    assert per_w % CHUNK == 0
    nchunks = per_w // CHUNK
    assert nchunks % 2 == 0 and nchunks >= 6
    mesh = plsc.VectorSubcoreMesh(core_axis_name="c", subcore_axis_name="s")

    @functools.partial(
        pl.kernel,
        mesh=mesh,
        out_type=jax.ShapeDtypeStruct((N, D), jnp.float32),
        scratch_types=[
            pltpu.VMEM((per_w,), jnp.int32),
            pltpu.VMEM((S, D), jnp.float32),
            pltpu.VMEM((2, CHUNK, D), jnp.float32),
            pltpu.VMEM((2, CHUNK, D), jnp.float32),
            pltpu.SemaphoreType.DMA,
            pltpu.SemaphoreType.DMA,
            pltpu.SemaphoreType.DMA,
            pltpu.SemaphoreType.DMA,
        ],
    )
    def emb_kernel(idx_hbm, table_hbm, pe_hbm, out_hbm,
                   idx_v, pe_v, gbuf, obuf, g0, g1, o0, o1):
        wid = lax.axis_index("s") * NC + lax.axis_index("c")
        base = wid * per_w
        gsem = (g0, g1)
        osem = (o0, o1)

        pltpu.sync_copy(idx_hbm.at[pl.ds(base, per_w)], idx_v)

        def start_gather(c, b):
            pltpu.async_copy(
                table_hbm.at[idx_v.at[pl.ds(c * CHUNK, CHUNK)]],
                gbuf.at[b], gsem[b])

        def start_out(c, b):
            pltpu.async_copy(
                obuf.at[b], out_hbm.at[pl.ds(base + c * CHUNK, CHUNK)], osem[b])

        def wait_gather(b):
            pltpu.make_async_copy(out_hbm.at[pl.ds(base, CHUNK)],
                                  gbuf.at[b], gsem[b]).wait()

        def wait_out(b):
            pltpu.make_async_copy(obuf.at[b],
                                  out_hbm.at[pl.ds(base, CHUNK)], osem[b]).wait()

        def add_pe(c, b):
            row0 = base + c * CHUNK
            s0 = row0 // B
            m = jnp.minimum((s0 + 1) * B - row0, CHUNK)
            s1 = jnp.minimum(s0 + 1, S - 1)

            def add_run(rlo, rhi, s_fixed):
                pes = [pe_v[s_fixed, pl.ds(j * L, L)] for j in range(D // L)]

                @plsc.parallel_loop(rlo, rhi, unroll=2)
                def row_body(r):
                    for j in range(D // L):
                        sl = pl.ds(j * L, L)
                        obuf[b, r, sl] = gbuf[b, r, sl] + pes[j]

            add_run(0, m, s0)
            add_run(m, CHUNK, s1)

        start_gather(0, 0)
        start_gather(1, 1)
        pltpu.sync_copy(pe_hbm, pe_v)

        for b in range(2):
            wait_gather(b)
            add_pe(b, b)
            start_gather(b + 2, b)
            start_out(b, b)

        def outer_body(o, carry):
            for b in range(2):
                c = 2 * o + b
                wait_gather(b)
                wait_out(b)
                add_pe(c, b)
                start_gather(c + 2, b)
                start_out(c, b)
            return carry

        lax.fori_loop(1, nchunks // 2 - 1, outer_body, 0)

        for b in range(2):
            c = nchunks - 2 + b
            wait_gather(b)
            wait_out(b)
            add_pe(c, b)
            start_out(c, b)

        wait_out(0)
        wait_out(1)

    return emb_kernel


def kernel(x, token_table):
    B, S = x.shape
    D = token_table.shape[1]
    pe = _pos_encoding(S, D)
    idx = x.T.reshape(-1)
    out = _make_sc_kernel(B, S, D)(idx, token_table, pe)
    return out.reshape(S, B, D).transpose(1, 0, 2)


# speedup vs baseline: 1.0608x; 1.0608x over previous
import functools

import jax
import jax.numpy as jnp
import numpy as np
from jax import lax
from jax.experimental import pallas as pl
from jax.experimental.pallas import tpu as pltpu
from jax.experimental.pallas import tpu_sc as plsc


def _pos_encoding(seq_len, d_model):
    pos = np.arange(seq_len, dtype=np.float32)[:, None]
    two_i = np.arange(0, d_model, 2, dtype=np.float32)
    div = np.power(np.float32(10000.0), two_i / np.float32(d_model))
    enc = np.zeros((seq_len, d_model), dtype=np.float32)
    enc[:, 0::2] = np.sin(pos / div)
    enc[:, 1::2] = np.cos(pos / div)
    return jnp.asarray(enc)


@functools.lru_cache(maxsize=None)
def _make_sc_kernel(B, S, D):
    info = plsc.get_sparse_core_info()
    NC, NS, L = info.num_cores, info.num_subcores, info.num_lanes
    NW = NC * NS
    N = B * S
    assert N % NW == 0
    per_w = N // NW
    CHUNK = 40
    assert per_w % CHUNK == 0
    nchunks = per_w // CHUNK
    assert nchunks % 2 == 0 and nchunks >= 6
    mesh = plsc.VectorSubcoreMesh(core_axis_name="c", subcore_axis_name="s")

    @functools.partial(
        pl.kernel,
        mesh=mesh,
        out_type=jax.ShapeDtypeStruct((N, D), jnp.float32),
        scratch_types=[
            pltpu.VMEM((per_w,), jnp.int32),
            pltpu.VMEM((S, D), jnp.float32),
            pltpu.VMEM((2, CHUNK, D), jnp.float32),
            pltpu.VMEM((2, CHUNK, D), jnp.float32),
            pltpu.SemaphoreType.DMA,
            pltpu.SemaphoreType.DMA,
            pltpu.SemaphoreType.DMA,
            pltpu.SemaphoreType.DMA,
        ],
    )
    def emb_kernel(idx_hbm, table_hbm, pe_hbm, out_hbm,
                   idx_v, pe_v, gbuf, obuf, g0, g1, o0, o1):
        wid = lax.axis_index("s") * NC + lax.axis_index("c")
        base = wid * per_w
        gsem = (g0, g1)
        osem = (o0, o1)

        pltpu.sync_copy(idx_hbm.at[pl.ds(base, per_w)], idx_v)

        def start_gather(c, b):
            pltpu.async_copy(
                table_hbm.at[idx_v.at[pl.ds(c * CHUNK, CHUNK)]],
                gbuf.at[b], gsem[b])

        def start_out(c, b):
            pltpu.async_copy(
                obuf.at[b], out_hbm.at[pl.ds(base + c * CHUNK, CHUNK)], osem[b])

        def wait_gather(b):
            pltpu.make_async_copy(out_hbm.at[pl.ds(base, CHUNK)],
                                  gbuf.at[b], gsem[b]).wait()

        def wait_out(b):
            pltpu.make_async_copy(obuf.at[b],
                                  out_hbm.at[pl.ds(base, CHUNK)], osem[b]).wait()

        def add_pe(c, b):
            row0 = base + c * CHUNK
            s0 = row0 // B
            m = jnp.minimum((s0 + 1) * B - row0, CHUNK)
            s1 = jnp.minimum(s0 + 1, S - 1)

            def add_run(rlo, rhi, s_fixed):
                pes = [pe_v[s_fixed, pl.ds(j * L, L)] for j in range(D // L)]

                @plsc.parallel_loop(rlo, rhi)
                def row_body(r):
                    for j in range(D // L):
                        sl = pl.ds(j * L, L)
                        obuf[b, r, sl] = gbuf[b, r, sl] + pes[j]

            add_run(0, m, s0)
            add_run(m, CHUNK, s1)

        start_gather(0, 0)
        start_gather(1, 1)
        pltpu.sync_copy(pe_hbm, pe_v)

        for b in range(2):
            wait_gather(b)
            add_pe(b, b)
            start_gather(b + 2, b)
            start_out(b, b)

        def outer_body(o, carry):
            for b in range(2):
                c = 2 * o + b
                wait_gather(b)
                wait_out(b)
                add_pe(c, b)
                start_gather(c + 2, b)
                start_out(c, b)
            return carry

        lax.fori_loop(1, nchunks // 2 - 1, outer_body, 0)

        for b in range(2):
            c = nchunks - 2 + b
            wait_gather(b)
            wait_out(b)
            add_pe(c, b)
            start_out(c, b)

        wait_out(0)
        wait_out(1)

    return emb_kernel


def kernel(x, token_table):
    B, S = x.shape
    D = token_table.shape[1]
    pe = _pos_encoding(S, D)
    idx = x.T.reshape(-1)
    out = _make_sc_kernel(B, S, D)(idx, token_table, pe)
    return out.reshape(S, B, D).transpose(1, 0, 2)
